# baseline (speedup 1.0000x reference)
import functools

import jax
import jax.numpy as jnp
from jax import lax
from jax.experimental import pallas as pl
from jax.experimental.pallas import tpu as pltpu
from jax.experimental.pallas import tpu_sc as plsc

POINCARE_EPS = 1e-05
MAX_PROJ_NORM = 5.0

NC = 2
NS = 16
NW = NC * NS
L = 16

B = 4096 * 200
D = 64
SUB = 128
NSUB = 4
CHUNK = SUB * NSUB
ROWS_PER_W = B // NW
NCHUNKS = ROWS_PER_W // CHUNK
NPAIRS = NCHUNKS // 2

def _lanesum(s):
    lanes = lax.iota(jnp.int32, L)
    dnums = lax.GatherDimensionNumbers(
        offset_dims=(), collapsed_slice_dims=(0,), start_index_map=(0,))
    for step in (8, 4, 2, 1):
        perm = (lanes ^ step)[:, None]
        s = s + lax.gather(s, perm, dnums, slice_sizes=(1,),
                           mode=lax.GatherScatterMode.PROMISE_IN_BOUNDS)
    return s


def _rsqrt(x):
    r = jnp.full((L,), 1.0, dtype=jnp.float32)
    for k in (64, 32, 16, 8, 4, 2):
        t = jnp.float32(2.0 ** k)
        pred = x >= t
        x = jnp.where(pred, x * jnp.float32(2.0 ** (-k)), x)
        r = jnp.where(pred, r * jnp.float32(2.0 ** (-k // 2)), r)
    y = 1.0 / x
    for _ in range(6):
        y = y * (1.5 - 0.5 * x * y * y)
    return y * r


@functools.partial(
    pl.kernel,
    out_type=jax.ShapeDtypeStruct((B, D), jnp.float32),
    mesh=plsc.VectorSubcoreMesh(core_axis_name="c", subcore_axis_name="s"),
    compiler_params=pltpu.CompilerParams(use_tc_tiling_on_sc=False),
    scratch_types=[
        pltpu.VMEM((NSUB, SUB), jnp.int32),
        pltpu.VMEM((NSUB, SUB), jnp.int32),
        pltpu.VMEM((CHUNK, D), jnp.float32),
        pltpu.VMEM((CHUNK, D), jnp.float32),
        pltpu.VMEM((L,), jnp.float32),
        pltpu.SemaphoreType.DMA,
        pltpu.SemaphoreType.DMA,
        pltpu.SemaphoreType.DMA,
        pltpu.SemaphoreType.DMA,
    ],
)
def _sc_gather_clamp(idx_hbm, table_hbm, scale_hbm, out_hbm,
                     idx0, idx1, rows0, rows1, scale_v,
                     g0, g1, o0, o1):
    idx_bufs = [idx0, idx1]
    row_bufs = [rows0, rows1]
    gsems = [g0, g1]
    osems = [o0, o1]

    wid = lax.axis_index("s") * NC + lax.axis_index("c")
    idx_base = wid * (ROWS_PER_W // SUB)
    row_base = wid * ROWS_PER_W

    pltpu.sync_copy(scale_hbm, scale_v)
    sv = scale_v[...]
    sv2 = sv * sv
    sv2_s = sv2[0]
    thresh = MAX_PROJ_NORM * MAX_PROJ_NORM

    def start(c, bi):
        pltpu.sync_copy(idx_hbm.at[pl.ds(idx_base + c * NSUB, NSUB), :],
                        idx_bufs[bi])
        for j in range(NSUB):
            pltpu.async_copy(table_hbm.at[idx_bufs[bi].at[j]],
                             row_bufs[bi].at[pl.ds(j * SUB, SUB), :],
                             gsems[bi])

    def wait_gathers(bi):
        pltpu.make_async_copy(out_hbm.at[pl.ds(0, CHUNK), :],
                              row_bufs[bi], gsems[bi]).wait()

    def wait_outcopy(bi):
        pltpu.make_async_copy(row_bufs[bi],
                              out_hbm.at[pl.ds(0, CHUNK), :], osems[bi]).wait()

    def compute(bi):
        rows = row_bufs[bi]

        def row_body(r, carry):
            v = [rows[r, pl.ds(k * L, L)] for k in range(4)]
            s = v[0] * v[0] + v[1] * v[1] + v[2] * v[2] + v[3] * v[3]
            ssum = _lanesum(s)
            big = ssum[0] * sv2_s > thresh

            @pl.when(big)
            def _():
                n2 = ssum * sv2
                f = jnp.where(n2 > thresh,
                              MAX_PROJ_NORM * _rsqrt(n2),
                              jnp.full((L,), 1.0, dtype=jnp.float32)) * sv
                for k in range(4):
                    rows[r, pl.ds(k * L, L)] = v[k] * f

            @pl.when(jnp.logical_not(big))
            def _():
                for k in range(4):
                    rows[r, pl.ds(k * L, L)] = v[k] * sv

            return carry

        lax.fori_loop(0, CHUNK, row_body, 0)

    def process(c, bi):
        nb = 1 - bi
        nxt = c + 1

        @pl.when(nxt < NCHUNKS)
        def _():
            @pl.when(c >= 1)
            def _():
                wait_outcopy(nb)
            start(nxt, nb)

        wait_gathers(bi)
        compute(bi)
        pltpu.async_copy(row_bufs[bi],
                         out_hbm.at[pl.ds(row_base + c * CHUNK, CHUNK), :],
                         osems[bi])

    start(0, 0)

    def pair_body(g, carry):
        process(2 * g, 0)
        process(2 * g + 1, 1)
        return carry

    lax.fori_loop(0, NPAIRS, pair_body, 0)
    wait_outcopy(0)
    wait_outcopy(1)


def kernel(token_ids, embed_weight, scale):
    idx = token_ids.reshape(B // SUB, SUB).astype(jnp.int32)
    scale_vec = jnp.broadcast_to(scale.astype(jnp.float32), (L,))
    out = _sc_gather_clamp(idx, embed_weight, scale_vec)
    return out.reshape(token_ids.shape + (D,))
